# baseline (speedup 1.0000x reference)
import functools

import jax
import jax.numpy as jnp
from jax import lax
from jax.experimental import pallas as pl
from jax.experimental.pallas import tpu as pltpu
from jax.experimental.pallas import tpu_sc as plsc

IDX_W = 128
SUB = 256
LANES = 16


@functools.lru_cache(maxsize=None)
def _make_gather(h_tot, b_tot, v, d):
    info = plsc.get_sparse_core_info()
    nw = info.num_cores * info.num_subcores
    b_per_w = b_tot // nw
    q_tot = b_per_w // SUB
    k = SUB // IDX_W
    assert b_tot % nw == 0 and b_per_w % SUB == 0 and SUB % IDX_W == 0
    assert q_tot == 2, "pipeline below is written for 2 sub-blocks/row"
    mesh = plsc.VectorSubcoreMesh(core_axis_name="c", subcore_axis_name="s")

    @functools.partial(
        pl.kernel,
        mesh=mesh,
        out_type=jax.ShapeDtypeStruct((h_tot, d, b_tot), jnp.float32),
        compiler_params=pltpu.CompilerParams(
            use_tc_tiling_on_sc=False, needs_layout_passes=False
        ),
        scratch_types=[
            pltpu.VMEM((h_tot, b_per_w), jnp.int32),
            pltpu.VMEM((2, SUB, d), jnp.float32),
            pltpu.VMEM((2, d, SUB), jnp.float32),
            pltpu.SemaphoreType.DMA,
            pltpu.SemaphoreType.DMA,
            pltpu.SemaphoreType.DMA,
            pltpu.SemaphoreType.DMA,
        ],
    )
    def gather(xt_hbm, table_hbm, out_hbm, idx_v, rows_v, trans_v,
               gsem0, gsem1, ssem0, ssem1):
        wid = lax.axis_index("s") * info.num_cores + lax.axis_index("c")
        b0 = wid * b_per_w
        gsems = (gsem0, gsem1)
        ssems = (ssem0, ssem1)

        pltpu.sync_copy(xt_hbm.at[:, pl.ds(b0, b_per_w)], idx_v)

        def fire_gather(slot, hh, q):
            for j in range(k):
                pltpu.async_copy(
                    table_hbm.at[idx_v.at[hh, pl.ds(q * SUB + j * IDX_W, IDX_W)]],
                    rows_v.at[slot, pl.ds(j * IDX_W, IDX_W)],
                    gsems[slot],
                )

        def wait_gather(slot, hh, q):
            for j in range(k):
                pltpu.make_async_copy(
                    table_hbm.at[idx_v.at[hh, pl.ds(q * SUB + j * IDX_W, IDX_W)]],
                    rows_v.at[slot, pl.ds(j * IDX_W, IDX_W)],
                    gsems[slot],
                ).wait()

        def out_view(slot, hh):
            return out_hbm.at[hh, :, pl.ds(b0 + slot * SUB, SUB)]

        def fire_store(slot, hh):
            pltpu.async_copy(trans_v.at[slot], out_view(slot, hh), ssems[slot])

        def wait_store(slot, hh):
            pltpu.make_async_copy(
                trans_v.at[slot], out_view(slot, hh), ssems[slot]
            ).wait()

        lane = lax.iota(jnp.int32, LANES)

        def transpose(slot):
            def col(c, carry):
                cvec = jnp.full((LANES,), c, dtype=jnp.int32)
                for j in range(SUB // LANES):
                    val = plsc.load_gather(
                        rows_v.at[slot], [j * LANES + lane, cvec]
                    )
                    trans_v[slot, c, pl.ds(j * LANES, LANES)] = val
                return carry

            lax.fori_loop(0, d, col, 0, unroll=False)

        def step(slot, hh, first, fire_next):
            wait_gather(slot, hh, slot)
            if not first:
                wait_store(slot, hh - 1)
            transpose(slot)
            fire_store(slot, hh)
            if fire_next:
                fire_gather(slot, hh + 1, slot)

        for slot in range(2):
            fire_gather(slot, 0, slot)
        for slot in range(2):
            step(slot, 0, True, True)

        def body(i, carry):
            for slot in range(2):
                step(slot, i, False, True)
            return carry

        lax.fori_loop(1, h_tot - 1, body, 0, unroll=False)
        for slot in range(2):
            step(slot, h_tot - 1, False, False)
        for slot in range(2):
            wait_store(slot, h_tot - 1)

    return gather


def kernel(x, table):
    b, h = x.shape
    v, d = table.shape
    xt = x.T.astype(jnp.int32)
    out = _make_gather(h, b, v, d)(xt, table)
    return out.transpose(2, 0, 1)


# speedup vs baseline: 1.4674x; 1.4674x over previous
import functools

import jax
import jax.numpy as jnp
from jax import lax
from jax.experimental import pallas as pl
from jax.experimental.pallas import tpu as pltpu
from jax.experimental.pallas import tpu_sc as plsc

IDX_W = 128
SUB = 256
LANES = 16


@functools.lru_cache(maxsize=None)
def _make_gather(h_tot, b_tot, v, d):
    info = plsc.get_sparse_core_info()
    nw = info.num_cores * info.num_subcores
    b_per_w = b_tot // nw
    q_tot = b_per_w // SUB
    k = SUB // IDX_W
    assert b_tot % nw == 0 and b_per_w % SUB == 0 and SUB % IDX_W == 0
    assert q_tot == 2, "pipeline below is written for 2 sub-blocks/row"
    mesh = plsc.VectorSubcoreMesh(core_axis_name="c", subcore_axis_name="s")

    @functools.partial(
        pl.kernel,
        mesh=mesh,
        out_type=jax.ShapeDtypeStruct((h_tot, d, b_tot), jnp.float32),
        compiler_params=pltpu.CompilerParams(
            use_tc_tiling_on_sc=False, needs_layout_passes=False
        ),
        scratch_types=[
            pltpu.VMEM((h_tot, b_per_w), jnp.int32),
            pltpu.VMEM((2, SUB, d), jnp.float32),
            pltpu.VMEM((2, d, SUB), jnp.float32),
            pltpu.SemaphoreType.DMA,
            pltpu.SemaphoreType.DMA,
            pltpu.SemaphoreType.DMA,
            pltpu.SemaphoreType.DMA,
        ],
    )
    def gather(xt_hbm, table_hbm, out_hbm, idx_v, rows_v, trans_v,
               gsem0, gsem1, ssem0, ssem1):
        wid = lax.axis_index("s") * info.num_cores + lax.axis_index("c")
        b0 = wid * b_per_w
        gsems = (gsem0, gsem1)
        ssems = (ssem0, ssem1)

        pltpu.sync_copy(xt_hbm.at[:, pl.ds(b0, b_per_w)], idx_v)

        def fire_gather(slot, hh, q):
            for j in range(k):
                pltpu.async_copy(
                    table_hbm.at[idx_v.at[hh, pl.ds(q * SUB + j * IDX_W, IDX_W)]],
                    rows_v.at[slot, pl.ds(j * IDX_W, IDX_W)],
                    gsems[slot],
                )

        def wait_gather(slot, hh, q):
            for j in range(k):
                pltpu.make_async_copy(
                    table_hbm.at[idx_v.at[hh, pl.ds(q * SUB + j * IDX_W, IDX_W)]],
                    rows_v.at[slot, pl.ds(j * IDX_W, IDX_W)],
                    gsems[slot],
                ).wait()

        def out_view(slot, hh):
            return out_hbm.at[hh, :, pl.ds(b0 + slot * SUB, SUB)]

        def fire_store(slot, hh):
            pltpu.async_copy(trans_v.at[slot], out_view(slot, hh), ssems[slot])

        def wait_store(slot, hh):
            pltpu.make_async_copy(
                trans_v.at[slot], out_view(slot, hh), ssems[slot]
            ).wait()

        lane = lax.iota(jnp.int32, LANES)
        perms = [(lane + t) % LANES for t in range(LANES)]

        def transpose(slot):
            def blk(j, carry):
                row = j * LANES + lane
                for c0 in range(0, d, LANES):
                    for t in range(LANES):
                        col = c0 + perms[t]
                        val = plsc.load_gather(rows_v.at[slot], [row, col])
                        plsc.store_scatter(trans_v.at[slot], [col, row], val)
                return carry

            lax.fori_loop(0, SUB // LANES, blk, 0, unroll=False)

        def step(slot, hh, first, fire_next):
            wait_gather(slot, hh, slot)
            if not first:
                wait_store(slot, hh - 1)
            transpose(slot)
            fire_store(slot, hh)
            if fire_next:
                fire_gather(slot, hh + 1, slot)

        for slot in range(2):
            fire_gather(slot, 0, slot)
        for slot in range(2):
            step(slot, 0, True, True)

        def body(i, carry):
            for slot in range(2):
                step(slot, i, False, True)
            return carry

        lax.fori_loop(1, h_tot - 1, body, 0, unroll=False)
        for slot in range(2):
            step(slot, h_tot - 1, False, False)
        for slot in range(2):
            wait_store(slot, h_tot - 1)

    return gather


def kernel(x, table):
    b, h = x.shape
    v, d = table.shape
    xt = x.T.astype(jnp.int32)
    out = _make_gather(h, b, v, d)(xt, table)
    return out.transpose(2, 0, 1)


# speedup vs baseline: 1.6202x; 1.1042x over previous
import functools

import jax
import jax.numpy as jnp
from jax import lax
from jax.experimental import pallas as pl
from jax.experimental.pallas import tpu as pltpu
from jax.experimental.pallas import tpu_sc as plsc

IDX_W = 128
NSLOT = 4


@functools.lru_cache(maxsize=None)
def _make_gather(n_tot, v, d):
    info = plsc.get_sparse_core_info()
    nw = info.num_cores * info.num_subcores
    n_per_w = n_tot // nw
    t_tot = n_per_w // IDX_W
    assert n_tot % nw == 0 and n_per_w % IDX_W == 0 and t_tot % NSLOT == 0
    mesh = plsc.VectorSubcoreMesh(core_axis_name="c", subcore_axis_name="s")

    @functools.partial(
        pl.kernel,
        mesh=mesh,
        out_type=jax.ShapeDtypeStruct((n_tot, d), jnp.float32),
        compiler_params=pltpu.CompilerParams(
            use_tc_tiling_on_sc=False, needs_layout_passes=False
        ),
        scratch_types=[
            pltpu.VMEM((t_tot, IDX_W), jnp.int32),
            pltpu.VMEM((NSLOT, IDX_W, d), jnp.float32),
        ]
        + [pltpu.SemaphoreType.DMA] * (2 * NSLOT),
    )
    def gather(xf_hbm, table_hbm, out_hbm, idx_v, rows_v, *sems):
        gsems = sems[:NSLOT]
        ssems = sems[NSLOT:]
        wid = lax.axis_index("s") * info.num_cores + lax.axis_index("c")
        r0 = wid * n_per_w

        pltpu.sync_copy(xf_hbm.at[pl.ds(wid * t_tot, t_tot)], idx_v)

        def fire_gather(t, slot):
            pltpu.async_copy(
                table_hbm.at[idx_v.at[t]], rows_v.at[slot], gsems[slot]
            )

        def wait_gather(t, slot):
            pltpu.make_async_copy(
                table_hbm.at[idx_v.at[t]], rows_v.at[slot], gsems[slot]
            ).wait()

        def out_view(t):
            return out_hbm.at[pl.ds(r0 + t * IDX_W, IDX_W)]

        def fire_store(t, slot):
            pltpu.async_copy(rows_v.at[slot], out_view(t), ssems[slot])

        def wait_store(t, slot):
            pltpu.make_async_copy(rows_v.at[slot], out_view(t), ssems[slot]).wait()

        for slot in range(NSLOT):
            fire_gather(slot, slot)

        def body(r, carry):
            t0 = r * NSLOT
            for slot in range(NSLOT):
                t = t0 + slot
                wait_gather(t, slot)
                fire_store(t, slot)
            for slot in range(NSLOT):
                t = t0 + slot
                wait_store(t, slot)
                fire_gather(t + NSLOT, slot)
            return carry

        lax.fori_loop(0, t_tot // NSLOT - 1, body, 0, unroll=False)

        t0 = t_tot - NSLOT
        for slot in range(NSLOT):
            wait_gather(t0 + slot, slot)
            fire_store(t0 + slot, slot)
        for slot in range(NSLOT):
            wait_store(t0 + slot, slot)

    return gather


def kernel(x, table):
    b, h = x.shape
    v, d = table.shape
    xf = x.reshape(b * h // IDX_W, IDX_W).astype(jnp.int32)
    out = _make_gather(b * h, v, d)(xf, table)
    return out.reshape(b, h, d)
